# baseline (speedup 1.0000x reference)
import functools

import jax
import jax.numpy as jnp
from jax import lax
from jax.experimental import pallas as pl
from jax.experimental.pallas import tpu as pltpu
from jax.experimental.pallas import tpu_sc as plsc

N = 10000
E = 320000
H = 128
G = 128
NUM_ATOM = 100
NUM_BOND = 10
NBP = 16
L = 4

NW = 16
CHUNK = 80
NCHUNK = E // NW // CHUNK
GCHUNK = 50
NGRP = NCHUNK // GCHUNK
NPAD = 10112
RPS = NPAD // 16
BLK = 1000


def _sc_body(aug_hbm, comb_hbm, dst_hbm, out_hbm,
             combg, dstg, r0, r1, r2, aggr_sh, gsem0, gsem1, gsem2, ssem):
    sid = lax.axis_index("s")

    zero16 = jnp.zeros((16,), jnp.float32)

    def _zrow(i, c):
        for k in range(H // 16):
            r0[i, pl.ds(k * 16, 16)] = zero16
        return c

    lax.fori_loop(0, CHUNK, _zrow, 0)
    for q in range(RPS // CHUNK):
        pltpu.sync_copy(r0, aggr_sh.at[pl.ds(sid * RPS + q * CHUNK, CHUNK)])
    rem = RPS % CHUNK
    if rem:
        pltpu.sync_copy(
            r0.at[pl.ds(0, rem)],
            aggr_sh.at[pl.ds(sid * RPS + (RPS // CHUNK) * CHUNK, rem)])
    plsc.subcore_barrier()

    bufs = (r0, r1, r2)
    gsems = (gsem0, gsem1, gsem2)

    def _group(g, c):
        pltpu.sync_copy(comb_hbm.at[sid, g], combg)
        pltpu.sync_copy(dst_hbm.at[sid, g], dstg)
        pltpu.async_copy(aug_hbm.at[combg.at[0]], r0, gsem0)
        pltpu.async_copy(aug_hbm.at[combg.at[1]], r1, gsem1)

        def _chunk(j, c2):
            def emit(cur, nxt, gcur, gnxt):
                pltpu.make_async_copy(
                    aug_hbm.at[combg.at[j]], cur, gcur).wait()

                @pl.when(j > 0)
                def _():
                    pltpu.make_async_copy(
                        nxt, aggr_sh.at[dstg.at[j - 1]], ssem).wait()

                @pl.when(j < GCHUNK - 2)
                def _():
                    pltpu.async_copy(aug_hbm.at[combg.at[j + 2]], nxt, gnxt)

                pltpu.async_copy(cur, aggr_sh.at[dstg.at[j]], ssem, add=True)

            for r in range(3):
                @pl.when(j % 3 == r)
                def _(r=r):
                    emit(bufs[r], bufs[(r + 2) % 3],
                         gsems[r], gsems[(r + 2) % 3])

            return c2

        lax.fori_loop(0, GCHUNK, _chunk, 0)
        pltpu.make_async_copy(
            bufs[(GCHUNK - 1) % 3],
            aggr_sh.at[dstg.at[GCHUNK - 1]], ssem).wait()
        return c

    lax.fori_loop(0, NGRP, _group, 0)
    plsc.subcore_barrier()

    r = sid * RPS
    pltpu.sync_copy(aggr_sh.at[pl.ds(r, RPS)],
                    out_hbm.at[pl.ds(r, RPS)])


def _sc_aggregate(aug, comb4, dst4):
    f = pl.kernel(
        _sc_body,
        out_type=jax.ShapeDtypeStruct((NPAD, H), jnp.float32),
        mesh=plsc.VectorSubcoreMesh(core_axis_name="c", subcore_axis_name="s",
                                    num_cores=1),
        scratch_types=[
            pltpu.VMEM((GCHUNK, CHUNK), jnp.int32),
            pltpu.VMEM((GCHUNK, CHUNK), jnp.int32),
            pltpu.VMEM((CHUNK, H), jnp.float32),
            pltpu.VMEM((CHUNK, H), jnp.float32),
            pltpu.VMEM((CHUNK, H), jnp.float32),
            pltpu.VMEM_SHARED((NPAD, H), jnp.float32),
            pltpu.SemaphoreType.DMA,
            pltpu.SemaphoreType.DMA,
            pltpu.SemaphoreType.DMA,
            pltpu.SemaphoreType.DMA,
        ],
    )
    return f(aug, comb4, dst4)


def _comb_body(src_ref, e_ref, out_ref):
    out_ref[...] = src_ref[...] * NBP + e_ref[...]


def _tc_comb(src2, e2):
    return pl.pallas_call(
        _comb_body,
        out_shape=jax.ShapeDtypeStruct((E // 128, 128), jnp.int32),
    )(src2, e2)


def _embed_body(h_ref, embh_ref, embe_ref, hx_ref, aug_ref):
    hb = h_ref[0, 0, :]
    onehot = (hb[:, None] == lax.broadcasted_iota(jnp.int32, (BLK, 128), 1))
    hx = jnp.dot(onehot.astype(jnp.float32), embh_ref[...],
                 preferred_element_type=jnp.float32)
    hx_ref[...] = hx
    aug_ref[...] = jax.nn.relu(hx[:, None, :] + embe_ref[...][None, :, :])


def _tc_embed(h3, embh_p, embe_p):
    return pl.pallas_call(
        _embed_body,
        grid=(N // BLK,),
        in_specs=[
            pl.BlockSpec((1, 1, BLK), lambda i: (i, 0, 0)),
            pl.BlockSpec((128, 128), lambda i: (0, 0)),
            pl.BlockSpec((NBP, H), lambda i: (0, 0)),
        ],
        out_specs=[
            pl.BlockSpec((BLK, H), lambda i: (i, 0)),
            pl.BlockSpec((BLK, NBP, H), lambda i: (i, 0, 0)),
        ],
        out_shape=[
            jax.ShapeDtypeStruct((N, H), jnp.float32),
            jax.ShapeDtypeStruct((N, NBP, H), jnp.float32),
        ],
    )(h3, embh_p, embe_p)


def _mlp_core(hx_ref, aggr_ref, w1_ref, b1_ref, w2_ref, b2_ref):
    z = hx_ref[...] + aggr_ref[...]
    t = jax.nn.relu(jnp.dot(z, w1_ref[...],
                            preferred_element_type=jnp.float32) + b1_ref[...])
    return jnp.dot(t, w2_ref[...],
                   preferred_element_type=jnp.float32) + b2_ref[...]


def _mlp_aug_body(hx_ref, aggr_ref, w1_ref, b1_ref, w2_ref, b2_ref, embe_ref,
                  nh_ref, aug_ref):
    nh = _mlp_core(hx_ref, aggr_ref, w1_ref, b1_ref, w2_ref, b2_ref)
    nh_ref[...] = nh
    aug_ref[...] = jax.nn.relu(nh[:, None, :] + embe_ref[...][None, :, :])


def _mlp_last_body(hx_ref, aggr_ref, w1_ref, b1_ref, w2_ref, b2_ref, nh_ref):
    nh_ref[...] = _mlp_core(hx_ref, aggr_ref, w1_ref, b1_ref, w2_ref, b2_ref)


def _tc_mlp(hx, aggr, w1, b1, w2, b2, embe_p, with_aug):
    wspec = pl.BlockSpec((128, 128), lambda i: (0, 0))
    bspec = pl.BlockSpec((1, 128), lambda i: (0, 0))
    in_specs = [
        pl.BlockSpec((BLK, H), lambda i: (i, 0)),
        pl.BlockSpec((BLK, H), lambda i: (i, 0)),
        wspec, bspec, wspec, bspec,
    ]
    args = [hx, aggr, w1, b1, w2, b2]
    if with_aug:
        in_specs.append(pl.BlockSpec((NBP, H), lambda i: (0, 0)))
        args.append(embe_p)
        return pl.pallas_call(
            _mlp_aug_body,
            grid=(N // BLK,),
            in_specs=in_specs,
            out_specs=[
                pl.BlockSpec((BLK, H), lambda i: (i, 0)),
                pl.BlockSpec((BLK, NBP, H), lambda i: (i, 0, 0)),
            ],
            out_shape=[
                jax.ShapeDtypeStruct((N, H), jnp.float32),
                jax.ShapeDtypeStruct((N, NBP, H), jnp.float32),
            ],
        )(*args)
    return pl.pallas_call(
        _mlp_last_body,
        grid=(N // BLK,),
        in_specs=in_specs,
        out_specs=pl.BlockSpec((BLK, H), lambda i: (i, 0)),
        out_shape=jax.ShapeDtypeStruct((N, H), jnp.float32),
    )(*args)


def _readout_body(gid_ref, hx_ref, wr0_ref, br0_ref, wr1_ref, br1_ref,
                  wr2_ref, br2_ref, out_ref, acc_ref, cnt_ref):
    i = pl.program_id(0)

    @pl.when(i == 0)
    def _():
        acc_ref[...] = jnp.zeros_like(acc_ref)
        cnt_ref[...] = jnp.zeros_like(cnt_ref)

    gid = gid_ref[0, 0, :]
    onehot = (gid[:, None] == lax.broadcasted_iota(jnp.int32, (BLK, G), 1)
              ).astype(jnp.float32)
    acc_ref[...] += lax.dot_general(
        onehot, hx_ref[...], (((0,), (0,)), ((), ())),
        preferred_element_type=jnp.float32)
    cnt_ref[...] += jnp.sum(onehot, axis=0, keepdims=True)

    @pl.when(i == pl.num_programs(0) - 1)
    def _():
        hg = acc_ref[...] / jnp.maximum(cnt_ref[...], 1.0).reshape(G, 1)
        y = jax.nn.relu(jnp.dot(hg, wr0_ref[...],
                                preferred_element_type=jnp.float32)
                        + br0_ref[...])
        y = jax.nn.relu(jnp.dot(y, wr1_ref[...],
                                preferred_element_type=jnp.float32)
                        + br1_ref[...])
        y2 = jnp.sum(y * wr2_ref[...], axis=1, keepdims=True)
        out_ref[...] = y2 + br2_ref[...]


def _tc_readout(g3, hx, wr0p, br0p, wr1p, br1p, wr2p, br2p):
    wspec = pl.BlockSpec((128, 128), lambda i: (0, 0))
    bspec = pl.BlockSpec((1, 128), lambda i: (0, 0))
    return pl.pallas_call(
        _readout_body,
        grid=(N // BLK,),
        in_specs=[
            pl.BlockSpec((1, 1, BLK), lambda i: (i, 0, 0)),
            pl.BlockSpec((BLK, H), lambda i: (i, 0)),
            wspec, bspec, wspec, bspec, bspec, bspec,
        ],
        out_specs=pl.BlockSpec((G, 128), lambda i: (0, 0)),
        out_shape=jax.ShapeDtypeStruct((G, 128), jnp.float32),
        scratch_shapes=[
            pltpu.VMEM((G, G), jnp.float32),
            pltpu.VMEM((1, G), jnp.float32),
        ],
    )(g3, hx, wr0p, br0p, wr1p, br1p, wr2p, br2p)


def kernel(h, e, p, snorm_n, edge_index, graph_ids, emb_h, emb_e,
           W1s, b1s, W2s, b2s, Wr0, br0, Wr1, br1, Wr2, br2):
    f32 = jnp.float32
    src2 = edge_index[0].astype(jnp.int32).reshape(E // 128, 128)
    e2 = e.astype(jnp.int32).reshape(E // 128, 128)
    comb4 = _tc_comb(src2, e2).reshape(NW, NGRP, GCHUNK, CHUNK)
    dst4 = edge_index[1].astype(jnp.int32).reshape(NW, NGRP, GCHUNK, CHUNK)
    h3 = h.astype(jnp.int32).reshape(N // BLK, 1, BLK)
    g3 = graph_ids.astype(jnp.int32).reshape(N // BLK, 1, BLK)

    embh_p = jnp.zeros((128, 128), f32).at[:NUM_ATOM].set(emb_h)
    embe_p = jnp.zeros((NBP, H), f32).at[:NUM_BOND].set(emb_e)

    wr0p = jnp.zeros((128, 128), f32).at[:, :H // 2].set(Wr0)
    br0p = jnp.zeros((1, 128), f32).at[0, :H // 2].set(br0)
    wr1p = jnp.zeros((128, 128), f32).at[:H // 2, :H // 4].set(Wr1)
    br1p = jnp.zeros((1, 128), f32).at[0, :H // 4].set(br1)
    wr2p = jnp.zeros((1, 128), f32).at[0, :H // 4].set(Wr2[:, 0])
    br2p = jnp.broadcast_to(br2.reshape(1, 1), (1, 128)).astype(f32)

    hx, aug = _tc_embed(h3, embh_p, embe_p)
    for l in range(L):
        aggr = _sc_aggregate(aug.reshape(N * NBP, H), comb4, dst4)
        b1 = b1s[l].reshape(1, H)
        b2 = b2s[l].reshape(1, H)
        if l < L - 1:
            hx, aug = _tc_mlp(hx, aggr, W1s[l], b1, W2s[l], b2, embe_p, True)
        else:
            hx = _tc_mlp(hx, aggr, W1s[l], b1, W2s[l], b2, None, False)

    y = _tc_readout(g3, hx, wr0p, br0p, wr1p, br1p, wr2p, br2p)
    return y[:, :1]


# speedup vs baseline: 1.0414x; 1.0414x over previous
import functools

import jax
import jax.numpy as jnp
from jax import lax
from jax.experimental import pallas as pl
from jax.experimental.pallas import tpu as pltpu
from jax.experimental.pallas import tpu_sc as plsc

N = 10000
E = 320000
H = 128
G = 128
NUM_ATOM = 100
NUM_BOND = 10
NBP = 16
L = 4

NW = 16
CHUNK = 80
NCHUNK = E // NW // CHUNK
GCHUNK = 50
NGRP = NCHUNK // GCHUNK
NPAD = 10112
RPS = NPAD // 16
BLK = 1000


def _sc_body(aug_hbm, comb_hbm, dst_hbm, out_hbm,
             combg, dstg, r0, r1, r2, aggr_sh, gsem0, gsem1, gsem2, ssem):
    sid = lax.axis_index("s")

    zero16 = jnp.zeros((16,), jnp.float32)

    def _zrow(i, c):
        for k in range(H // 16):
            r0[i, pl.ds(k * 16, 16)] = zero16
        return c

    lax.fori_loop(0, CHUNK, _zrow, 0)
    for q in range(RPS // CHUNK):
        pltpu.sync_copy(r0, aggr_sh.at[pl.ds(sid * RPS + q * CHUNK, CHUNK)])
    rem = RPS % CHUNK
    if rem:
        pltpu.sync_copy(
            r0.at[pl.ds(0, rem)],
            aggr_sh.at[pl.ds(sid * RPS + (RPS // CHUNK) * CHUNK, rem)])
    plsc.subcore_barrier()

    bufs = (r0, r1, r2)
    gsems = (gsem0, gsem1, gsem2)

    def _group(g, c):
        pltpu.sync_copy(comb_hbm.at[sid, g], combg)
        pltpu.sync_copy(dst_hbm.at[sid, g], dstg)
        pltpu.async_copy(aug_hbm.at[combg.at[0]], r0, gsem0)
        pltpu.async_copy(aug_hbm.at[combg.at[1]], r1, gsem1)

        def _chunk(j, c2):
            def emit(cur, nxt, gcur, gnxt):
                pltpu.make_async_copy(
                    aug_hbm.at[combg.at[j]], cur, gcur).wait()

                @pl.when(j > 0)
                def _():
                    pltpu.make_async_copy(
                        nxt, aggr_sh.at[dstg.at[j - 1]], ssem).wait()

                @pl.when(j < GCHUNK - 2)
                def _():
                    pltpu.async_copy(aug_hbm.at[combg.at[j + 2]], nxt, gnxt)

                pltpu.async_copy(cur, aggr_sh.at[dstg.at[j]], ssem, add=True)

            for r in range(3):
                @pl.when(j % 3 == r)
                def _(r=r):
                    emit(bufs[r], bufs[(r + 2) % 3],
                         gsems[r], gsems[(r + 2) % 3])

            return c2

        lax.fori_loop(0, GCHUNK, _chunk, 0)
        pltpu.make_async_copy(
            bufs[(GCHUNK - 1) % 3],
            aggr_sh.at[dstg.at[GCHUNK - 1]], ssem).wait()
        return c

    lax.fori_loop(0, NGRP, _group, 0)
    plsc.subcore_barrier()

    r = sid * RPS
    pltpu.sync_copy(aggr_sh.at[pl.ds(r, RPS)],
                    out_hbm.at[pl.ds(r, RPS)])


def _sc_aggregate(aug, comb4, dst4):
    f = pl.kernel(
        _sc_body,
        out_type=jax.ShapeDtypeStruct((NPAD, H), jnp.float32),
        mesh=plsc.VectorSubcoreMesh(core_axis_name="c", subcore_axis_name="s",
                                    num_cores=1),
        scratch_types=[
            pltpu.VMEM((GCHUNK, CHUNK), jnp.int32),
            pltpu.VMEM((GCHUNK, CHUNK), jnp.int32),
            pltpu.VMEM((CHUNK, H), jnp.float32),
            pltpu.VMEM((CHUNK, H), jnp.float32),
            pltpu.VMEM((CHUNK, H), jnp.float32),
            pltpu.VMEM_SHARED((NPAD, H), jnp.float32),
            pltpu.SemaphoreType.DMA,
            pltpu.SemaphoreType.DMA,
            pltpu.SemaphoreType.DMA,
            pltpu.SemaphoreType.DMA,
        ],
    )
    return f(aug, comb4, dst4)


def _comb_body(src_ref, e_ref, out_ref):
    out_ref[...] = e_ref[...] * N + src_ref[...]


def _tc_comb(src2, e2):
    return pl.pallas_call(
        _comb_body,
        out_shape=jax.ShapeDtypeStruct((E // 128, 128), jnp.int32),
    )(src2, e2)


def _embed_body(h_ref, embh_ref, embe_ref, hx_ref, aug_ref):
    hb = h_ref[0, 0, :]
    onehot = (hb[:, None] == lax.broadcasted_iota(jnp.int32, (BLK, 128), 1))
    hx = jnp.dot(onehot.astype(jnp.float32), embh_ref[...],
                 preferred_element_type=jnp.float32)
    hx_ref[...] = hx
    aug_ref[...] = jax.nn.relu(embe_ref[...][:NUM_BOND, None, :]
                               + hx[None, :, :])


def _tc_embed(h3, embh_p, embe_p):
    return pl.pallas_call(
        _embed_body,
        grid=(N // BLK,),
        in_specs=[
            pl.BlockSpec((1, 1, BLK), lambda i: (i, 0, 0)),
            pl.BlockSpec((128, 128), lambda i: (0, 0)),
            pl.BlockSpec((NBP, H), lambda i: (0, 0)),
        ],
        out_specs=[
            pl.BlockSpec((BLK, H), lambda i: (i, 0)),
            pl.BlockSpec((NUM_BOND, BLK, H), lambda i: (0, i, 0)),
        ],
        out_shape=[
            jax.ShapeDtypeStruct((N, H), jnp.float32),
            jax.ShapeDtypeStruct((NUM_BOND, N, H), jnp.float32),
        ],
    )(h3, embh_p, embe_p)


def _mlp_core(hx_ref, aggr_ref, w1_ref, b1_ref, w2_ref, b2_ref):
    z = hx_ref[...] + aggr_ref[...]
    t = jax.nn.relu(jnp.dot(z, w1_ref[...],
                            preferred_element_type=jnp.float32) + b1_ref[...])
    return jnp.dot(t, w2_ref[...],
                   preferred_element_type=jnp.float32) + b2_ref[...]


def _mlp_aug_body(hx_ref, aggr_ref, w1_ref, b1_ref, w2_ref, b2_ref, embe_ref,
                  nh_ref, aug_ref):
    nh = _mlp_core(hx_ref, aggr_ref, w1_ref, b1_ref, w2_ref, b2_ref)
    nh_ref[...] = nh
    aug_ref[...] = jax.nn.relu(embe_ref[...][:NUM_BOND, None, :]
                               + nh[None, :, :])


def _mlp_last_body(hx_ref, aggr_ref, w1_ref, b1_ref, w2_ref, b2_ref, nh_ref):
    nh_ref[...] = _mlp_core(hx_ref, aggr_ref, w1_ref, b1_ref, w2_ref, b2_ref)


def _tc_mlp(hx, aggr, w1, b1, w2, b2, embe_p, with_aug):
    wspec = pl.BlockSpec((128, 128), lambda i: (0, 0))
    bspec = pl.BlockSpec((1, 128), lambda i: (0, 0))
    in_specs = [
        pl.BlockSpec((BLK, H), lambda i: (i, 0)),
        pl.BlockSpec((BLK, H), lambda i: (i, 0)),
        wspec, bspec, wspec, bspec,
    ]
    args = [hx, aggr, w1, b1, w2, b2]
    if with_aug:
        in_specs.append(pl.BlockSpec((NBP, H), lambda i: (0, 0)))
        args.append(embe_p)
        return pl.pallas_call(
            _mlp_aug_body,
            grid=(N // BLK,),
            in_specs=in_specs,
            out_specs=[
                pl.BlockSpec((BLK, H), lambda i: (i, 0)),
                pl.BlockSpec((NUM_BOND, BLK, H), lambda i: (0, i, 0)),
            ],
            out_shape=[
                jax.ShapeDtypeStruct((N, H), jnp.float32),
                jax.ShapeDtypeStruct((NUM_BOND, N, H), jnp.float32),
            ],
        )(*args)
    return pl.pallas_call(
        _mlp_last_body,
        grid=(N // BLK,),
        in_specs=in_specs,
        out_specs=pl.BlockSpec((BLK, H), lambda i: (i, 0)),
        out_shape=jax.ShapeDtypeStruct((N, H), jnp.float32),
    )(*args)


def _readout_body(gid_ref, hx_ref, wr0_ref, br0_ref, wr1_ref, br1_ref,
                  wr2_ref, br2_ref, out_ref, acc_ref, cnt_ref):
    i = pl.program_id(0)

    @pl.when(i == 0)
    def _():
        acc_ref[...] = jnp.zeros_like(acc_ref)
        cnt_ref[...] = jnp.zeros_like(cnt_ref)

    gid = gid_ref[0, 0, :]
    onehot = (gid[:, None] == lax.broadcasted_iota(jnp.int32, (BLK, G), 1)
              ).astype(jnp.float32)
    acc_ref[...] += lax.dot_general(
        onehot, hx_ref[...], (((0,), (0,)), ((), ())),
        preferred_element_type=jnp.float32)
    cnt_ref[...] += jnp.sum(onehot, axis=0, keepdims=True)

    @pl.when(i == pl.num_programs(0) - 1)
    def _():
        hg = acc_ref[...] / jnp.maximum(cnt_ref[...], 1.0).reshape(G, 1)
        y = jax.nn.relu(jnp.dot(hg, wr0_ref[...],
                                preferred_element_type=jnp.float32)
                        + br0_ref[...])
        y = jax.nn.relu(jnp.dot(y, wr1_ref[...],
                                preferred_element_type=jnp.float32)
                        + br1_ref[...])
        y2 = jnp.sum(y * wr2_ref[...], axis=1, keepdims=True)
        out_ref[...] = y2 + br2_ref[...]


def _tc_readout(g3, hx, wr0p, br0p, wr1p, br1p, wr2p, br2p):
    wspec = pl.BlockSpec((128, 128), lambda i: (0, 0))
    bspec = pl.BlockSpec((1, 128), lambda i: (0, 0))
    return pl.pallas_call(
        _readout_body,
        grid=(N // BLK,),
        in_specs=[
            pl.BlockSpec((1, 1, BLK), lambda i: (i, 0, 0)),
            pl.BlockSpec((BLK, H), lambda i: (i, 0)),
            wspec, bspec, wspec, bspec, bspec, bspec,
        ],
        out_specs=pl.BlockSpec((G, 128), lambda i: (0, 0)),
        out_shape=jax.ShapeDtypeStruct((G, 128), jnp.float32),
        scratch_shapes=[
            pltpu.VMEM((G, G), jnp.float32),
            pltpu.VMEM((1, G), jnp.float32),
        ],
    )(g3, hx, wr0p, br0p, wr1p, br1p, wr2p, br2p)


def kernel(h, e, p, snorm_n, edge_index, graph_ids, emb_h, emb_e,
           W1s, b1s, W2s, b2s, Wr0, br0, Wr1, br1, Wr2, br2):
    f32 = jnp.float32
    src2 = edge_index[0].astype(jnp.int32).reshape(E // 128, 128)
    e2 = e.astype(jnp.int32).reshape(E // 128, 128)
    comb4 = _tc_comb(src2, e2).reshape(NW, NGRP, GCHUNK, CHUNK)
    dst4 = edge_index[1].astype(jnp.int32).reshape(NW, NGRP, GCHUNK, CHUNK)
    h3 = h.astype(jnp.int32).reshape(N // BLK, 1, BLK)
    g3 = graph_ids.astype(jnp.int32).reshape(N // BLK, 1, BLK)

    embh_p = jnp.zeros((128, 128), f32).at[:NUM_ATOM].set(emb_h)
    embe_p = jnp.zeros((NBP, H), f32).at[:NUM_BOND].set(emb_e)

    wr0p = jnp.zeros((128, 128), f32).at[:, :H // 2].set(Wr0)
    br0p = jnp.zeros((1, 128), f32).at[0, :H // 2].set(br0)
    wr1p = jnp.zeros((128, 128), f32).at[:H // 2, :H // 4].set(Wr1)
    br1p = jnp.zeros((1, 128), f32).at[0, :H // 4].set(br1)
    wr2p = jnp.zeros((1, 128), f32).at[0, :H // 4].set(Wr2[:, 0])
    br2p = jnp.broadcast_to(br2.reshape(1, 1), (1, 128)).astype(f32)

    hx, aug = _tc_embed(h3, embh_p, embe_p)
    for l in range(L):
        aggr = _sc_aggregate(aug.reshape(NUM_BOND * N, H), comb4, dst4)
        b1 = b1s[l].reshape(1, H)
        b2 = b2s[l].reshape(1, H)
        if l < L - 1:
            hx, aug = _tc_mlp(hx, aggr, W1s[l], b1, W2s[l], b2, embe_p, True)
        else:
            hx = _tc_mlp(hx, aggr, W1s[l], b1, W2s[l], b2, None, False)

    y = _tc_readout(g3, hx, wr0p, br0p, wr1p, br1p, wr2p, br2p)
    return y[:, :1]
